# baseline (speedup 1.0000x reference)
import functools

import jax
import jax.numpy as jnp
from jax.experimental import pallas as pl

_NUM_RES = 2


def _rnd(x):
    return x.astype(jnp.bfloat16).astype(jnp.float32)


def _conv_vp_body(x_ref, w_ref, b_ref, *rest, taps, H, W, cin, kc, relu,
                  has_res, round_out, bh):
    if has_res:
        res_ref, o_ref = rest
    else:
        (o_ref,) = rest
    cout = w_ref.shape[-1]
    for h0 in range(0, H, bh):
        zpad = (jnp.zeros((bh * W, kc - cin), jnp.float32)
                if kc > cin else None)
        cols = []
        for (dy, dx, c0) in taps:
            cols.append(x_ref[0, h0 + dy:h0 + dy + bh, dx:dx + W,
                              c0:c0 + cin].reshape(bh * W, cin))
            if zpad is not None:
                cols.append(zpad)
        xt = _rnd(jnp.concatenate(cols, axis=1))
        acc = jax.lax.dot_general(xt, w_ref[...], (((1,), (0,)), ((), ())),
                                  preferred_element_type=jnp.float32)
        acc = acc + b_ref[0][None, :]
        if has_res:
            acc = acc + res_ref[0, h0:h0 + bh].reshape(bh * W, cout)
        if relu:
            acc = jnp.maximum(acc, 0.0)
        if round_out:
            acc = _rnd(acc)
        o_ref[0, h0:h0 + bh] = acc.reshape(bh, W, cout)


def _conv_vp(xp, w_k, b, taps, cin, kc, out_hw, relu, res=None,
             round_out=False):
    N, Hp, Wp, Ctot = xp.shape
    K, Cout = w_k.shape
    H, W = out_hw
    b2 = b.reshape(1, Cout)
    in_specs = [
        pl.BlockSpec((1, Hp, Wp, Ctot), lambda n: (n, 0, 0, 0)),
        pl.BlockSpec((K, Cout), lambda n: (0, 0)),
        pl.BlockSpec((1, Cout), lambda n: (0, 0)),
    ]
    args = [xp, w_k, b2]
    if res is not None:
        in_specs.append(pl.BlockSpec((1, H, W, Cout), lambda n: (n, 0, 0, 0)))
        args.append(res)
    bh = 1
    for d in range(H, 0, -1):
        if H % d == 0 and d * W * K * 4 <= 2 ** 22:
            bh = d
            break
    body = functools.partial(_conv_vp_body, taps=tuple(taps), H=H, W=W,
                             cin=cin, kc=kc, relu=relu,
                             has_res=res is not None, round_out=round_out,
                             bh=bh)
    return pl.pallas_call(
        body,
        grid=(N,),
        in_specs=in_specs,
        out_specs=pl.BlockSpec((1, H, W, Cout), lambda n: (n, 0, 0, 0)),
        out_shape=jax.ShapeDtypeStruct((N, H, W, Cout), jnp.float32),
    )(*args)


def _pad1(x):
    return jnp.pad(x, ((0, 0), (1, 1), (1, 1), (0, 0)))


def _kc_for(cin):
    return ((cin + 127) // 128) * 128


def _w3_padded(w, kc):
    co, ci = w.shape[0], w.shape[1]
    wt = jnp.transpose(w, (2, 3, 1, 0)).reshape(9, ci, co)
    return jnp.pad(wt, ((0, 0), (0, kc - ci), (0, 0))).reshape(9 * kc, co)


def _conv3(h, w, b, relu, res=None, round_out=False):
    H, W = h.shape[1], h.shape[2]
    ci = h.shape[3]
    kc = _kc_for(ci)
    taps = [(dy, dx, 0) for dy in range(3) for dx in range(3)]
    return _conv_vp(_pad1(h), _w3_padded(w, kc), b, taps, ci, kc, (H, W),
                    relu, res=res, round_out=round_out)


def _phase_cat(xp):
    n, h2, w2, c = xp.shape
    x = xp.reshape(n, h2 // 2, 2, w2 // 2, 2, c)
    x = jnp.transpose(x, (0, 1, 3, 2, 4, 5))
    return x.reshape(n, h2 // 2, w2 // 2, 4 * c)


def _strided_conv(h, w, b):
    ci = h.shape[3]
    xc = _phase_cat(_pad1(h))
    O = h.shape[1] // 2
    kc = 128 if ci >= 8 else ci
    taps = []
    wrows = []
    wt = jnp.transpose(w, (2, 3, 1, 0))
    for ky in range(4):
        for kx in range(4):
            p, q = ky % 2, kx % 2
            a, bb = ky // 2, kx // 2
            taps.append((a, bb, (p * 2 + q) * ci))
            wrows.append(jnp.pad(wt[ky, kx], ((0, kc - ci), (0, 0))))
    w_k = jnp.concatenate(wrows, axis=0)
    return _conv_vp(xc, w_k, b, taps, ci, kc, (O, O), relu=True)


def _taps_tconv(w, r, s):
    ci = w.shape[0]
    kc = _kc_for(ci)
    wt = jnp.transpose(w, (2, 3, 0, 1))
    rows = []
    for a in range(2):
        for bb in range(2):
            rows.append(jnp.pad(wt[3 - r - 2 * a, 3 - s - 2 * bb],
                                ((0, kc - ci), (0, 0))))
    return jnp.concatenate(rows, axis=0), kc


def _tconv_up(h, w, b, relu, round_out):
    xp = _pad1(h)
    H, W = h.shape[1], h.shape[2]
    ci, co = w.shape[0], w.shape[1]
    phases = []
    for r in range(2):
        row = []
        for s in range(2):
            w_k, kc = _taps_tconv(w, r, s)
            taps = [(r + a, s + bb, 0) for a in range(2) for bb in range(2)]
            row.append(_conv_vp(xp, w_k, b, taps, ci, kc, (H, W), relu=relu,
                                round_out=round_out))
        phases.append(jnp.stack(row, axis=0))
    y = jnp.stack(phases, axis=0)
    y = jnp.transpose(y, (2, 3, 0, 4, 1, 5))
    return y.reshape(h.shape[0], 2 * H, 2 * W, co)


def _residual_unit(h, w1, b1, w2, b2):
    a = _conv3(h, w1, b1, relu=True)
    return _conv3(a, w2, b2, relu=True, res=h)


def _vq_body(z_ref, cbt_ref, cb_ref, q_ref, l_ref):
    z = z_ref[...]
    cbt = cbt_ref[...]
    cb = cb_ref[...]
    cb2 = jnp.sum(cb * cb, axis=1)[None, :]
    zz = jnp.sum(z * z, axis=1, keepdims=True)
    g = jax.lax.dot_general(z, cbt, (((1,), (0,)), ((), ())),
                            preferred_element_type=jnp.float32)
    d = (zz - 2.0 * g) + cb2
    dmin = jnp.min(d, axis=1, keepdims=True)
    K = d.shape[1]
    iota = jax.lax.broadcasted_iota(jnp.int32, d.shape, 1)
    idx = jnp.min(jnp.where(d <= dmin, iota, K), axis=1, keepdims=True)
    oh = (iota == idx).astype(jnp.float32)
    q = jax.lax.dot_general(oh, cb, (((1,), (0,)), ((), ())),
                            preferred_element_type=jnp.float32,
                            precision=jax.lax.Precision.HIGHEST)
    q_ref[...] = z + (q - z)
    dif = q - z
    l_ref[...] = jnp.sum(dif * dif).reshape(1, 1, 1)


def _vq(zf, codebook, bm):
    M, C = zf.shape
    K = codebook.shape[0]
    cbt = jnp.transpose(codebook)
    nblk = M // bm
    qst, lparts = pl.pallas_call(
        _vq_body,
        grid=(nblk,),
        in_specs=[
            pl.BlockSpec((bm, C), lambda i: (i, 0)),
            pl.BlockSpec((C, K), lambda i: (0, 0)),
            pl.BlockSpec((K, C), lambda i: (0, 0)),
        ],
        out_specs=[
            pl.BlockSpec((bm, C), lambda i: (i, 0)),
            pl.BlockSpec((1, 1, 1), lambda i: (i, 0, 0)),
        ],
        out_shape=[
            jax.ShapeDtypeStruct((M, C), jnp.float32),
            jax.ShapeDtypeStruct((nblk, 1, 1), jnp.float32),
        ],
    )(zf, cbt, codebook)
    return qst, lparts


def kernel(x, params, codebook):
    p = params
    h = jnp.transpose(x, (0, 2, 3, 1))

    for i in range(2):
        h = _strided_conv(h, p['enc%d_w' % i], p['enc%d_b' % i])
        for j in range(_NUM_RES):
            h = _residual_unit(h, p['enc%d_res%d_w1' % (i, j)],
                               p['enc%d_res%d_b1' % (i, j)],
                               p['enc%d_res%d_w2' % (i, j)],
                               p['enc%d_res%d_b2' % (i, j)])
    H, W = h.shape[1], h.shape[2]
    z = _conv3(h, p['enc_out_w'], p['enc_out_b'], relu=False,
               round_out=True)

    N = z.shape[0]
    C = z.shape[3]
    zf = z.reshape(N * H * W, C)
    M = zf.shape[0]
    bm = 1568 if M % 1568 == 0 else M
    qst, lparts = _vq(zf, codebook, bm=bm)
    loss = (1.25 / zf.size) * jnp.sum(lparts)
    qh = qst.reshape(N, H, W, C)

    h = _conv3(qh, p['dec_in_w'], p['dec_in_b'], relu=False)
    for i in range(2):
        for j in range(_NUM_RES):
            h = _residual_unit(h, p['dec%d_res%d_w1' % (i, j)],
                               p['dec%d_res%d_b1' % (i, j)],
                               p['dec%d_res%d_w2' % (i, j)],
                               p['dec%d_res%d_b2' % (i, j)])
        h = _tconv_up(h, p['dec%d_up_w' % i], p['dec%d_up_b' % i],
                      relu=(i == 0), round_out=False)
    recon = jnp.transpose(h, (0, 3, 1, 2))
    return recon, loss
